# Initial kernel scaffold; baseline (speedup 1.0000x reference)
import jax
import jax.numpy as jnp
from jax.experimental import pallas as pl


def kernel(token_ids, embedding):
    raise NotImplementedError("write your pallas kernel here")


# speedup vs baseline: 1.1024x; 1.1024x over previous
import functools

import jax
import jax.numpy as jnp
from jax import lax
from jax.experimental import pallas as pl
from jax.experimental.pallas import tpu as pltpu
from jax.experimental.pallas import tpu_sc as plsc

_NUM_CORES = 2
_NUM_SUBCORES = 16
_NW = _NUM_CORES * _NUM_SUBCORES

_D = 32
_B = 16384 * 50
_B_PER_W = _B // _NW
_CHUNK = 1600
_N_CHUNKS = _B_PER_W // _CHUNK


def _gather_body(table_hbm, idx_hbm, out_hbm, idx_v, rows_v, sem):
    wid = lax.axis_index("s") * _NUM_CORES + lax.axis_index("c")
    base = wid * _B_PER_W

    def step(i, carry):
        off = base + i * _CHUNK
        pltpu.sync_copy(idx_hbm.at[pl.ds(off, _CHUNK)], idx_v)
        pltpu.async_copy(table_hbm.at[idx_v], rows_v, sem).wait()
        pltpu.sync_copy(rows_v, out_hbm.at[pl.ds(off, _CHUNK)])
        return carry

    lax.fori_loop(0, _N_CHUNKS, step, 0)


@jax.jit
def _gather(token_ids_flat, embedding):
    mesh = plsc.VectorSubcoreMesh(core_axis_name="c", subcore_axis_name="s")
    run = pl.kernel(
        _gather_body,
        out_type=jax.ShapeDtypeStruct((_B, _D), jnp.float32),
        mesh=mesh,
        scratch_types=[
            pltpu.VMEM((_CHUNK,), jnp.int32),
            pltpu.VMEM((_CHUNK, _D), jnp.float32),
            pltpu.SemaphoreType.DMA,
        ],
        compiler_params=pltpu.CompilerParams(use_tc_tiling_on_sc=False),
    )
    return run(embedding, token_ids_flat)


def kernel(token_ids, embedding):
    flat = token_ids.reshape(-1).astype(jnp.int32)
    out = _gather(flat, embedding)
    return out.reshape(token_ids.shape + (embedding.shape[1],))


# speedup vs baseline: 1.1096x; 1.0065x over previous
import jax
import jax.numpy as jnp
from jax import lax
from jax.experimental import pallas as pl
from jax.experimental.pallas import tpu as pltpu
from jax.experimental.pallas import tpu_sc as plsc

_NUM_CORES = 2
_NUM_SUBCORES = 16
_NW = _NUM_CORES * _NUM_SUBCORES

_D = 32
_B = 16384 * 50
_B_PER_W = _B // _NW
_CHUNK = 1600
_N_CHUNKS = _B_PER_W // _CHUNK


def _gather_body(table_hbm, idx_hbm, out_hbm, idx_v, rows_v,
                 sem_i0, sem_i1, sem_g0, sem_g1, sem_o0, sem_o1):
    wid = lax.axis_index("s") * _NUM_CORES + lax.axis_index("c")
    base = wid * _B_PER_W
    sem_i = (sem_i0, sem_i1)
    sem_g = (sem_g0, sem_g1)
    sem_o = (sem_o0, sem_o1)

    def start_idx(i):
        off = base + i * _CHUNK
        return pltpu.async_copy(
            idx_hbm.at[pl.ds(off, _CHUNK)], idx_v.at[i % 2], sem_i[i % 2])

    def start_gather(i):
        return pltpu.async_copy(
            table_hbm.at[idx_v.at[i % 2]], rows_v.at[i % 2], sem_g[i % 2])

    def start_out(i):
        off = base + i * _CHUNK
        return pltpu.async_copy(
            rows_v.at[i % 2], out_hbm.at[pl.ds(off, _CHUNK)], sem_o[i % 2])

    n = _N_CHUNKS
    c_idx = [None] * n
    c_g = [None] * n
    c_o = [None] * n

    c_idx[0] = start_idx(0)
    if n > 1:
        c_idx[1] = start_idx(1)
    c_idx[0].wait()
    c_g[0] = start_gather(0)

    for i in range(n):
        c_g[i].wait()
        c_o[i] = start_out(i)
        if i + 2 < n:
            c_idx[i + 2] = start_idx(i + 2)
        if i + 1 < n:
            c_idx[i + 1].wait()
            if i >= 1:
                c_o[i - 1].wait()
            c_g[i + 1] = start_gather(i + 1)

    if n > 1:
        c_o[n - 2].wait()
    c_o[n - 1].wait()


@jax.jit
def _gather(token_ids_flat, embedding):
    mesh = plsc.VectorSubcoreMesh(core_axis_name="c", subcore_axis_name="s")
    run = pl.kernel(
        _gather_body,
        out_type=jax.ShapeDtypeStruct((_B, _D), jnp.float32),
        mesh=mesh,
        scratch_types=[
            pltpu.VMEM((2, _CHUNK), jnp.int32),
            pltpu.VMEM((2, _CHUNK, _D), jnp.float32),
            pltpu.SemaphoreType.DMA,
            pltpu.SemaphoreType.DMA,
            pltpu.SemaphoreType.DMA,
            pltpu.SemaphoreType.DMA,
            pltpu.SemaphoreType.DMA,
            pltpu.SemaphoreType.DMA,
        ],
        compiler_params=pltpu.CompilerParams(use_tc_tiling_on_sc=False),
    )
    return run(embedding, token_ids_flat)


def kernel(token_ids, embedding):
    flat = token_ids.reshape(-1).astype(jnp.int32)
    out = _gather(flat, embedding)
    return out.reshape(token_ids.shape + (embedding.shape[1],))
